# baseline (speedup 1.0000x reference)
import functools

import jax
import jax.numpy as jnp
from jax import lax
from jax.experimental import pallas as pl
from jax.experimental.pallas import tpu as pltpu
from jax.experimental.pallas import tpu_sc as plsc

B, T, D = 2, 2048, 2048
NC, NS, L = 2, 16, 16
NW = NC * NS
ROWS_PER_W = T // NW
R = 8
CHUNK = R * D
N_CHUNKS = ROWS_PER_W // R

_mesh = plsc.VectorSubcoreMesh(core_axis_name="c", subcore_axis_name="s")


@functools.partial(
    pl.kernel,
    mesh=_mesh,
    out_type=jax.ShapeDtypeStruct((B * T * D,), jnp.float32),
    scratch_types=[
        pltpu.VMEM((CHUNK,), jnp.float32),
        pltpu.VMEM((CHUNK,), jnp.float32),
        pltpu.VMEM((CHUNK,), jnp.float32),
    ],
)
def _sc_add(x_hbm, pe_hbm, out_hbm, x0_v, x1_v, pe_v):
    wid = lax.axis_index("s") * NC + lax.axis_index("c")
    base = wid * (ROWS_PER_W * D)

    def chunk_body(ci, carry):
        off = base + ci * CHUNK
        pltpu.sync_copy(pe_hbm.at[pl.ds(off, CHUNK)], pe_v)
        pltpu.sync_copy(x_hbm.at[pl.ds(off, CHUNK)], x0_v)
        pltpu.sync_copy(x_hbm.at[pl.ds(T * D + off, CHUNK)], x1_v)

        def add_body(i, c):
            pe16 = pe_v[pl.ds(i * L, L)]
            plsc.addupdate(x0_v.at[pl.ds(i * L, L)], pe16)
            plsc.addupdate(x1_v.at[pl.ds(i * L, L)], pe16)
            return c

        lax.fori_loop(0, CHUNK // L, add_body, 0, unroll=4)
        pltpu.sync_copy(x0_v, out_hbm.at[pl.ds(off, CHUNK)])
        pltpu.sync_copy(x1_v, out_hbm.at[pl.ds(T * D + off, CHUNK)])
        return carry

    lax.fori_loop(0, N_CHUNKS, chunk_body, 0)


def kernel(x, pe):
    out_flat = _sc_add(x.reshape(-1), pe.reshape(-1))
    return out_flat.reshape(B, T, D)
